# baseline (speedup 1.0000x reference)
import jax
import jax.numpy as jnp
from jax.experimental import pallas as pl
from jax.experimental.pallas import tpu as pltpu

N = 100000
CH = 128

_SCHEDULE = [1000, 2000, 4000, 8000] + [14000] * 5 + [8000, 4000, 2000, 1000]
assert sum(_SCHEDULE) == N
_STARTS = [sum(_SCHEDULE[:k]) for k in range(len(_SCHEDULE))]
_MAXC = max(_SCHEDULE)


def _body(x_hbm, w_ref, o_hbm, inb, outb, in_sem, out_sem):
    w = w_ref[...].astype(jnp.bfloat16)
    nchunks = len(_SCHEDULE)

    def in_copy(k):
        s, sz = _STARTS[k], _SCHEDULE[k]
        return pltpu.make_async_copy(
            x_hbm.at[pl.ds(s, sz), :], inb.at[k % 2, pl.ds(0, sz), :],
            in_sem.at[k % 2])

    def out_copy(k):
        s, sz = _STARTS[k], _SCHEDULE[k]
        return pltpu.make_async_copy(
            outb.at[k % 2, pl.ds(0, sz), :], o_hbm.at[pl.ds(s, sz), :],
            out_sem.at[k % 2])

    in_copy(0).start()
    in_copy(1).start()
    for k in range(nchunks):
        in_copy(k).wait()
        if k >= 2:
            out_copy(k - 2).wait()
        sz = _SCHEDULE[k]
        outb[k % 2, pl.ds(0, sz), :] = jax.lax.dot_general(
            inb[k % 2, pl.ds(0, sz), :].astype(jnp.bfloat16), w,
            dimension_numbers=(((1,), (1,)), ((), ())),
            preferred_element_type=jnp.float32)
        out_copy(k).start()
        if k + 2 < nchunks:
            in_copy(k + 2).start()
    out_copy(nchunks - 2).wait()
    out_copy(nchunks - 1).wait()


def kernel(x_src, W):
    return pl.pallas_call(
        _body,
        in_specs=[
            pl.BlockSpec(memory_space=pl.ANY),
            pl.BlockSpec(memory_space=pltpu.MemorySpace.VMEM),
        ],
        out_specs=pl.BlockSpec(memory_space=pl.ANY),
        out_shape=jax.ShapeDtypeStruct((N, CH), jnp.float32),
        scratch_shapes=[
            pltpu.VMEM((2, _MAXC, CH), jnp.float32),
            pltpu.VMEM((2, _MAXC, CH), jnp.float32),
            pltpu.SemaphoreType.DMA((2,)),
            pltpu.SemaphoreType.DMA((2,)),
        ],
    )(x_src, W)


# speedup vs baseline: 1.1828x; 1.1828x over previous
import jax
import jax.numpy as jnp
from jax.experimental import pallas as pl
from jax.experimental.pallas import tpu as pltpu

N = 100000
CH = 128
NBUF = 4

_SCHEDULE = [1000, 2000, 4000, 8000] + [12000] * 6 + [8000, 4000, 1000]
assert sum(_SCHEDULE) == N
_STARTS = [sum(_SCHEDULE[:k]) for k in range(len(_SCHEDULE))]
_MAXC = max(_SCHEDULE)


def _body(x_hbm, w_ref, o_hbm, inb, outb, in_sem, out_sem):
    w = w_ref[...].astype(jnp.bfloat16)
    nchunks = len(_SCHEDULE)

    def in_copy(k):
        s, sz = _STARTS[k], _SCHEDULE[k]
        return pltpu.make_async_copy(
            x_hbm.at[pl.ds(s, sz), :], inb.at[k % NBUF, pl.ds(0, sz), :],
            in_sem.at[k % NBUF])

    def out_copy(k):
        s, sz = _STARTS[k], _SCHEDULE[k]
        return pltpu.make_async_copy(
            outb.at[k % NBUF, pl.ds(0, sz), :], o_hbm.at[pl.ds(s, sz), :],
            out_sem.at[k % NBUF])

    for k in range(NBUF - 1):
        in_copy(k).start()
    for k in range(nchunks):
        in_copy(k).wait()
        if k + NBUF - 1 < nchunks:
            in_copy(k + NBUF - 1).start()
        if k >= NBUF:
            out_copy(k - NBUF).wait()
        sz = _SCHEDULE[k]
        outb[k % NBUF, pl.ds(0, sz), :] = jax.lax.dot_general(
            inb[k % NBUF, pl.ds(0, sz), :].astype(jnp.bfloat16), w,
            dimension_numbers=(((1,), (1,)), ((), ())),
            preferred_element_type=jnp.float32)
        out_copy(k).start()
    for k in range(max(0, nchunks - NBUF), nchunks):
        out_copy(k).wait()


def kernel(x_src, W):
    return pl.pallas_call(
        _body,
        in_specs=[
            pl.BlockSpec(memory_space=pl.ANY),
            pl.BlockSpec(memory_space=pltpu.MemorySpace.VMEM),
        ],
        out_specs=pl.BlockSpec(memory_space=pl.ANY),
        out_shape=jax.ShapeDtypeStruct((N, CH), jnp.float32),
        scratch_shapes=[
            pltpu.VMEM((NBUF, _MAXC, CH), jnp.float32),
            pltpu.VMEM((NBUF, _MAXC, CH), jnp.float32),
            pltpu.SemaphoreType.DMA((NBUF,)),
            pltpu.SemaphoreType.DMA((NBUF,)),
        ],
        compiler_params=pltpu.CompilerParams(
            vmem_limit_bytes=112 * 1024 * 1024,
        ),
    )(x_src, W)


# speedup vs baseline: 1.1890x; 1.0053x over previous
import jax
import jax.numpy as jnp
from jax.experimental import pallas as pl
from jax.experimental.pallas import tpu as pltpu

N = 100000
CH = 128
BLOCK_N = 16000


def _matmul_kernel(x_ref, w_ref, o_ref):
    o_ref[...] = jax.lax.dot_general(
        x_ref[...].astype(jnp.bfloat16), w_ref[...].astype(jnp.bfloat16),
        dimension_numbers=(((1,), (1,)), ((), ())),
        preferred_element_type=jnp.float32,
    )


def kernel(x_src, W):
    grid = (pl.cdiv(N, BLOCK_N),)
    return pl.pallas_call(
        _matmul_kernel,
        grid=grid,
        in_specs=[
            pl.BlockSpec((BLOCK_N, CH), lambda i: (i, 0)),
            pl.BlockSpec((CH, CH), lambda i: (0, 0)),
        ],
        out_specs=pl.BlockSpec((BLOCK_N, CH), lambda i: (i, 0)),
        out_shape=jax.ShapeDtypeStruct((N, CH), jnp.float32),
        compiler_params=pltpu.CompilerParams(
            dimension_semantics=("parallel",),
        ),
    )(x_src, W)
